# Initial kernel scaffold; baseline (speedup 1.0000x reference)
import jax
import jax.numpy as jnp
from jax.experimental import pallas as pl


def kernel(x, edge_index, W_in, b_in, W_gcn, b_gcn, ln_g, ln_b, W_out, b_out):
    raise NotImplementedError("write your pallas kernel here")


# speedup vs baseline: 6.8272x; 6.8272x over previous
import functools

import jax
import jax.numpy as jnp
import numpy as np
from jax import lax
from jax.experimental import pallas as pl
from jax.experimental.pallas import tpu as pltpu
from jax.experimental.pallas import tpu_sc as plsc

N = 50000
E = 800000
DF = 64
DH = 128
BN = 1000
GRID = N // BN

NS = 16
EB = 80
RS = 12544
NPAD = 4 * RS
RPT = RS // NS
WROWS = 112
WITER = RPT // WROWS

_MESH = plsc.VectorSubcoreMesh(
    core_axis_name="c", subcore_axis_name="s", num_cores=2, num_subcores=NS
)


def _sc_agg(y, srcb, dstb, bounds):
    @functools.partial(
        pl.kernel,
        out_type=jax.ShapeDtypeStruct((NPAD, DH), jnp.float32),
        mesh=_MESH,
        scratch_types=[
            pltpu.VMEM((EB,), jnp.int32),
            pltpu.VMEM((EB,), jnp.int32),
            pltpu.VMEM((EB, DH), jnp.float32),
            pltpu.VMEM((WROWS, DH), jnp.float32),
            pltpu.VMEM((16,), jnp.int32),
            pltpu.VMEM_SHARED((RS + 8, DH), jnp.float32),
        ],
        compiler_params=pltpu.CompilerParams(needs_layout_passes=False),
    )
    def agg_kernel(y_hbm, src_hbm, dst_hbm, b_hbm, out,
                   sidx, didx, rows, zbuf, bnd, acc):
        cid = lax.axis_index("c")
        sid = lax.axis_index("s")

        pltpu.sync_copy(b_hbm, bnd)
        bvec = bnd[...]
        lanes = lax.iota(jnp.int32, 16)

        def zb_body(i, _):
            for q in range(DH // 16):
                zbuf[i, pl.ds(q * 16, 16)] = jnp.zeros((16,), jnp.float32)
            return _

        lax.fori_loop(0, WROWS, zb_body, 0)

        def run_pass(q):
            base = q * RS
            lo = jnp.sum(jnp.where(lanes == q, bvec, 0))
            hi = jnp.sum(jnp.where(lanes == q + 1, bvec, 0))
            lo80 = (lo // EB) * EB
            nb = (hi + (EB - 1) - lo80) // EB

            def zero_body(j, _):
                pltpu.sync_copy(zbuf, acc.at[pl.ds(sid * RPT + j * WROWS, WROWS)])
                return _

            lax.fori_loop(0, WITER, zero_body, 0)
            plsc.subcore_barrier()

            nt = (nb - sid + NS - 1) // NS

            def edge_body(k, _):
                off = lo80 + (k * NS + sid) * EB
                pltpu.sync_copy(src_hbm.at[pl.ds(off, EB)], sidx)
                pltpu.sync_copy(dst_hbm.at[pl.ds(off, EB)], didx)
                pltpu.sync_copy(y_hbm.at[sidx], rows)
                for j in range(EB // 16):
                    d = didx[pl.ds(j * 16, 16)]
                    t = d - base
                    ok = (t >= 0) & (t < RS)
                    didx[pl.ds(j * 16, 16)] = jnp.where(ok, t, RS)
                pltpu.sync_copy(rows, acc.at[didx], add=True)
                return _

            lax.fori_loop(0, nt, edge_body, 0)
            plsc.subcore_barrier()

            rbase = sid * RPT

            def wb(j, _):
                r = rbase + j * WROWS
                pltpu.sync_copy(acc.at[pl.ds(r, WROWS)],
                                out.at[pl.ds(base + r, WROWS)])
                return _

            lax.fori_loop(0, WITER, wb, 0)
            plsc.subcore_barrier()

        @pl.when(cid == 0)
        def _():
            run_pass(0)
            run_pass(1)

        @pl.when(cid == 1)
        def _():
            run_pass(2)
            run_pass(3)

    return agg_kernel(y, srcb, dstb, bounds)


def _row_spec(w):
    return pl.BlockSpec((BN, w), lambda i: (i, 0))


def _const_spec(shape):
    return pl.BlockSpec(shape, lambda i: (0,) * len(shape))


def _tc_input(x, degfull, W_in, b_in):
    def body(x_ref, dg, Wi, bi, h_ref, dinv_ref):
        h_ref[...] = (
            jnp.dot(x_ref[...], Wi[...], preferred_element_type=jnp.float32)
            + bi[...]
        )
        deg = dg[:, 0:1] + 1.0
        dinv_ref[...] = lax.rsqrt(deg)

    return pl.pallas_call(
        body,
        grid=(GRID,),
        in_specs=[
            _row_spec(DF),
            _row_spec(DH),
            _const_spec((DF, DH)),
            _const_spec((1, DH)),
        ],
        out_specs=[_row_spec(DH), _row_spec(1)],
        out_shape=[
            jax.ShapeDtypeStruct((N, DH), jnp.float32),
            jax.ShapeDtypeStruct((N, 1), jnp.float32),
        ],
    )(x, degfull, W_in, b_in)


def _tc_y(h, dinv, W_gcn):
    def body(h_ref, dinv_ref, W, y_ref):
        y_ref[...] = (
            jnp.dot(h_ref[...], W[...], preferred_element_type=jnp.float32)
            * dinv_ref[...]
        )

    return pl.pallas_call(
        body,
        grid=(GRID,),
        in_specs=[_row_spec(DH), _row_spec(1), _const_spec((DH, DH))],
        out_specs=_row_spec(DH),
        out_shape=jax.ShapeDtypeStruct((N, DH), jnp.float32),
    )(h, dinv, W_gcn)


def _gcn_update(h_ref, a_ref, y_ref, dinv, bg, lg, lb, dt):
    dh = (a_ref[...] + y_ref[...]) * dinv[...] + bg[...]
    mu = jnp.mean(dh, axis=1, keepdims=True)
    var = jnp.mean(jnp.square(dh - mu), axis=1, keepdims=True)
    dh = jnp.tanh((dh - mu) * lax.rsqrt(var + 1e-5) * lg[...] + lb[...])
    return h_ref[...] + dt * dh


def _tc_step(h, agg, y, dinv, b_gcn, ln_g, ln_b, W_gcn, dt):
    def body(h_ref, a_ref, y_ref, dinv_ref, bg, lg, lb, W, ho, yo):
        hn = _gcn_update(h_ref, a_ref, y_ref, dinv_ref, bg, lg, lb, dt)
        ho[...] = hn
        yo[...] = (
            jnp.dot(hn, W[...], preferred_element_type=jnp.float32)
            * dinv_ref[...]
        )

    return pl.pallas_call(
        body,
        grid=(GRID,),
        in_specs=[
            _row_spec(DH),
            _row_spec(DH),
            _row_spec(DH),
            _row_spec(1),
            _const_spec((1, DH)),
            _const_spec((1, DH)),
            _const_spec((1, DH)),
            _const_spec((DH, DH)),
        ],
        out_specs=[_row_spec(DH), _row_spec(DH)],
        out_shape=[
            jax.ShapeDtypeStruct((N, DH), jnp.float32),
            jax.ShapeDtypeStruct((N, DH), jnp.float32),
        ],
    )(h, agg, y, dinv, b_gcn, ln_g, ln_b, W_gcn)


def _tc_final(h, agg, y, dinv, b_gcn, ln_g, ln_b, W_out, b_out, dt):
    def body(h_ref, a_ref, y_ref, dinv_ref, bg, lg, lb, Wo, bo,
             out_ref, acc_ref):
        hn = _gcn_update(h_ref, a_ref, y_ref, dinv_ref, bg, lg, lb, dt)
        psum = jnp.sum(hn, axis=0, keepdims=True)
        i = pl.program_id(0)

        @pl.when(i == 0)
        def _():
            acc_ref[...] = psum

        @pl.when(i > 0)
        def _():
            acc_ref[...] = acc_ref[...] + psum

        @pl.when(i == GRID - 1)
        def _():
            out_ref[...] = (
                jnp.dot(acc_ref[...] * (1.0 / N), Wo[...],
                        preferred_element_type=jnp.float32)
                + bo[...]
            )

    return pl.pallas_call(
        body,
        grid=(GRID,),
        in_specs=[
            _row_spec(DH),
            _row_spec(DH),
            _row_spec(DH),
            _row_spec(1),
            _const_spec((1, DH)),
            _const_spec((1, DH)),
            _const_spec((1, DH)),
            _const_spec((DH, DH)),
            _const_spec((1, DH)),
        ],
        out_specs=pl.BlockSpec((1, DH), lambda i: (0, 0)),
        out_shape=jax.ShapeDtypeStruct((1, DH), jnp.float32),
        scratch_shapes=[pltpu.VMEM((1, DH), jnp.float32)],
    )(h, agg, y, dinv, b_gcn, ln_g, ln_b, W_out, b_out)


def kernel(x, edge_index, W_in, b_in, W_gcn, b_gcn, ln_g, ln_b, W_out, b_out):
    src = edge_index[0]
    dst = edge_index[1]

    bucket = dst // RS
    perm = jnp.argsort(bucket, stable=True)
    srcb = src[perm]
    dstb = dst[perm]
    counts = jnp.sum(jax.nn.one_hot(bucket, 4, dtype=jnp.int32), axis=0)
    bounds = jnp.concatenate(
        [jnp.zeros((1,), jnp.int32), jnp.cumsum(counts).astype(jnp.int32),
         jnp.full((11,), E, jnp.int32)]
    )[:16]

    degfull = _sc_agg(jnp.ones((N, DH), jnp.float32), srcb, dstb, bounds)
    h, dinv = _tc_input(x, degfull, W_in, b_in.reshape(1, DH))

    bg = b_gcn.reshape(1, DH)
    lg = ln_g.reshape(1, DH)
    lb = ln_b.reshape(1, DH)

    tt = np.linspace(0.0, 1.0, 10).astype(np.float32)
    dts = [float(tt[i] - tt[i - 1]) for i in range(1, 10)]

    y = _tc_y(h, dinv, W_gcn)
    for step in range(1, 10):
        agg = _sc_agg(y, srcb, dstb, bounds)
        dt = dts[step - 1]
        if step < 9:
            h, y = _tc_step(h, agg, y, dinv, bg, lg, lb, W_gcn, dt)
        else:
            out = _tc_final(h, agg, y, dinv, bg, lg, lb, W_out,
                            b_out.reshape(1, DH), dt)
    return out
